# baseline (speedup 1.0000x reference)
import functools

import jax
import jax.numpy as jnp
from jax import lax
from jax.experimental import pallas as pl
from jax.experimental.pallas import tpu as pltpu
from jax.experimental.pallas import tpu_sc as plsc

_VOCAB = 1000000
_EMB = 64
_TAG = 64
_BATCH = 1024
_SEQ = 200

_NTOK = _BATCH * _SEQ
_NW = 32
_PER_W = _NTOK // _NW
_CH = 128
_NCH = _PER_W // _CH


def _make_sc_gather():
    mesh = plsc.VectorSubcoreMesh(core_axis_name="c", subcore_axis_name="s")

    @functools.partial(
        pl.kernel,
        mesh=mesh,
        out_type=jax.ShapeDtypeStruct((_NTOK, _EMB), jnp.float32),
        scratch_types=[
            pltpu.VMEM((_NCH, _CH), jnp.int32),
            pltpu.VMEM((_CH, _EMB), jnp.float32),
            pltpu.SemaphoreType.DMA,
        ],
        compiler_params=pltpu.CompilerParams(use_tc_tiling_on_sc=False),
    )
    def gather_kernel(table_hbm, idx_hbm, out_hbm, idx_v, rows_v, sem):
        wid = lax.axis_index("s") * 2 + lax.axis_index("c")
        base = wid * _PER_W
        pltpu.sync_copy(idx_hbm.at[wid], idx_v)

        def body(c, carry):
            pltpu.async_copy(table_hbm.at[idx_v.at[c]], rows_v, sem).wait()
            pltpu.sync_copy(rows_v, out_hbm.at[pl.ds(base + c * _CH, _CH)])
            return carry

        lax.fori_loop(0, _NCH, body, 0)

    return gather_kernel


_sc_gather = _make_sc_gather()

_MM_BLK = 2048


def _mm_body(x_ref, w_ref, b_ref, o_ref):
    o_ref[...] = (
        jnp.dot(x_ref[...], w_ref[...], preferred_element_type=jnp.float32)
        + b_ref[...]
    )


def _project(rows, W, b):
    grid = (_NTOK // _MM_BLK,)
    return pl.pallas_call(
        _mm_body,
        grid=grid,
        in_specs=[
            pl.BlockSpec((_MM_BLK, _EMB), lambda i: (i, 0)),
            pl.BlockSpec((_EMB, _TAG), lambda i: (0, 0)),
            pl.BlockSpec((1, _TAG), lambda i: (0, 0)),
        ],
        out_specs=pl.BlockSpec((_MM_BLK, _TAG), lambda i: (i, 0)),
        out_shape=jax.ShapeDtypeStruct((_NTOK, _TAG), jnp.float32),
    )(rows, W, b.reshape(1, _TAG))


def kernel(emb_table, W, b, batch_w, batch_x, batch_w_lengths, batch_x_lengths):
    idx = batch_x.reshape(_NW, _NCH, _CH).astype(jnp.int32)
    rows = _sc_gather(emb_table, idx)
    out = _project(rows, W, b)
    return out.reshape(_BATCH, _SEQ, _TAG)
